# baseline (speedup 1.0000x reference)
import functools

import jax
import jax.numpy as jnp
from jax import lax
from jax.experimental import pallas as pl
from jax.experimental.pallas import tpu as pltpu
from jax.experimental.pallas import tpu_sc as plsc

_LANES = 16
_SUB = 8
_BLK = 512


@functools.lru_cache(maxsize=None)
def _build_sc_lookup(n_batch: int, n_seq: int, n_rows: int, n_cols: int):
    info = plsc.get_sparse_core_info()
    num_cores, num_subcores = info.num_cores, info.num_subcores
    n_workers = num_cores * num_subcores

    assert n_seq % _SUB == 0 and n_batch % _BLK == 0
    n_stiles = n_seq // _SUB
    n_bblks = n_batch // _BLK
    n_units = n_stiles * n_bblks
    assert n_units % n_workers == 0
    units_per_w = n_units // n_workers
    groups = _SUB * _BLK // _LANES
    grp_per_row = _BLK // _LANES

    mesh = plsc.VectorSubcoreMesh(core_axis_name="c", subcore_axis_name="s")

    @functools.partial(
        pl.kernel,
        mesh=mesh,
        out_type=jax.ShapeDtypeStruct((n_cols, n_seq, n_batch), jnp.float32),
        compiler_params=pltpu.CompilerParams(needs_layout_passes=False),
        scratch_types=[
            pltpu.VMEM((n_rows * n_cols,), jnp.float32),
            pltpu.VMEM((_SUB, _BLK), jnp.int32),
            pltpu.VMEM((_SUB, _BLK), jnp.int32),
            pltpu.VMEM((n_cols, _SUB, _BLK), jnp.float32),
            pltpu.VMEM((n_cols, _SUB, _BLK), jnp.float32),
            pltpu.SemaphoreType.DMA,
            pltpu.SemaphoreType.DMA,
            pltpu.SemaphoreType.DMA,
            pltpu.SemaphoreType.DMA,
        ],
    )
    def lookup(
        idx_hbm, tbl_hbm, out_hbm, tbl,
        idx_a, idx_b, out_a, out_b, isem_a, isem_b, osem_a, osem_b,
    ):
        wid = lax.axis_index("s") * num_cores + lax.axis_index("c")
        pltpu.sync_copy(tbl_hbm, tbl)
        u0 = wid * units_per_w
        bufs = ((idx_a, out_a, isem_a, osem_a), (idx_b, out_b, isem_b, osem_b))

        def idx_copy(u, ib, sem):
            s0 = (u // n_bblks) * _SUB
            b0 = (u % n_bblks) * _BLK
            return pltpu.make_async_copy(
                idx_hbm.at[pl.ds(s0, _SUB), pl.ds(b0, _BLK)], ib, sem
            )

        def out_copy(u, ob, sem):
            s0 = (u // n_bblks) * _SUB
            b0 = (u % n_bblks) * _BLK
            return pltpu.make_async_copy(
                ob, out_hbm.at[:, pl.ds(s0, _SUB), pl.ds(b0, _BLK)], sem
            )

        def compute(ib, ob):
            @plsc.parallel_loop(0, groups, unroll=16)
            def group_body(g):
                r = g // grp_per_row
                c = (g % grp_per_row) * _LANES
                iv = ib[r, pl.ds(c, _LANES)]
                for k in range(n_cols):
                    val = plsc.load_gather(tbl, [iv + k * n_rows])
                    ob[k, r, pl.ds(c, _LANES)] = val

        idx_copy(u0, idx_a, isem_a).start()
        idx_copy(u0 + 1, idx_b, isem_b).start()
        n_pairs = units_per_w // 2

        def pair_body(j, carry):
            for p, (ib, ob, isem, osem) in enumerate(bufs):
                i = u0 + 2 * j + p
                idx_copy(i, ib, isem).wait()

                @pl.when(j > 0)
                def _():
                    out_copy(i - 2, ob, osem).wait()

                compute(ib, ob)
                out_copy(i, ob, osem).start()

                @pl.when(2 * j + p + 2 < units_per_w)
                def _():
                    idx_copy(i + 2, ib, isem).start()

            return carry

        lax.fori_loop(0, n_pairs, pair_body, 0)

        if units_per_w % 2:
            last = u0 + units_per_w - 1
            idx_copy(last, idx_a, isem_a).wait()
            out_copy(last - 2, out_a, osem_a).wait()
            compute(idx_a, out_a)
            out_copy(last, out_a, osem_a).start()
            out_copy(last - 1, out_b, osem_b).wait()
            out_copy(last, out_a, osem_a).wait()
        else:
            out_copy(u0 + units_per_w - 2, out_a, osem_a).wait()
            out_copy(u0 + units_per_w - 1, out_b, osem_b).wait()

    return lookup


def kernel(atomic_numbers, weight):
    n_batch, n_seq = atomic_numbers.shape
    n_rows, n_cols = weight.shape
    lookup = _build_sc_lookup(n_batch, n_seq, n_rows, n_cols)
    idx_t = atomic_numbers.T
    tbl_t = weight.T.reshape(n_rows * n_cols)
    out_t = lookup(idx_t, tbl_t)
    return out_t.transpose(2, 1, 0)


# speedup vs baseline: 1.4336x; 1.4336x over previous
import functools

import jax
import jax.numpy as jnp
from jax import lax
from jax.experimental import pallas as pl
from jax.experimental.pallas import tpu as pltpu
from jax.experimental.pallas import tpu_sc as plsc

_LANES = 16
_SUB = 8
_BLK = 512


@functools.lru_cache(maxsize=None)
def _build_sc_lookup(n_batch: int, n_seq: int, n_rows: int, n_cols: int):
    info = plsc.get_sparse_core_info()
    num_cores, num_subcores = info.num_cores, info.num_subcores
    n_workers = num_cores * num_subcores

    assert n_seq % _SUB == 0 and n_batch % _BLK == 0
    n_stiles = n_seq // _SUB
    n_bblks = n_batch // _BLK
    n_units = n_stiles * n_bblks
    assert n_units % n_workers == 0
    units_per_w = n_units // n_workers
    groups = _SUB * _BLK // _LANES
    grp_per_row = _BLK // _LANES

    mesh = plsc.VectorSubcoreMesh(core_axis_name="c", subcore_axis_name="s")

    @functools.partial(
        pl.kernel,
        mesh=mesh,
        out_type=jax.ShapeDtypeStruct((n_cols, n_seq, n_batch), jnp.float32),
        compiler_params=pltpu.CompilerParams(needs_layout_passes=False),
        scratch_types=[
            pltpu.VMEM((n_rows * n_cols,), jnp.float32),
            pltpu.VMEM((_SUB, _BLK), jnp.int32),
            pltpu.VMEM((_SUB, _BLK), jnp.int32),
            pltpu.VMEM((n_cols, _SUB, _BLK), jnp.float32),
            pltpu.VMEM((n_cols, _SUB, _BLK), jnp.float32),
            pltpu.SemaphoreType.DMA,
            pltpu.SemaphoreType.DMA,
            pltpu.SemaphoreType.DMA,
            pltpu.SemaphoreType.DMA,
        ],
    )
    def lookup(
        idx_hbm, tbl_hbm, out_hbm, tbl,
        idx_a, idx_b, out_a, out_b, isem_a, isem_b, osem_a, osem_b,
    ):
        wid = lax.axis_index("s") * num_cores + lax.axis_index("c")
        pltpu.sync_copy(tbl_hbm, tbl)
        u0 = wid * units_per_w
        bufs = ((idx_a, out_a, isem_a, osem_a), (idx_b, out_b, isem_b, osem_b))

        def idx_copy(u, ib, sem):
            s0 = (u // n_bblks) * _SUB
            b0 = (u % n_bblks) * _BLK
            return pltpu.make_async_copy(
                idx_hbm.at[pl.ds(s0, _SUB), pl.ds(b0, _BLK)], ib, sem
            )

        def out_copy(u, ob, sem):
            s0 = (u // n_bblks) * _SUB
            b0 = (u % n_bblks) * _BLK
            return pltpu.make_async_copy(
                ob, out_hbm.at[:, pl.ds(s0, _SUB), pl.ds(b0, _BLK)], sem
            )

        def compute(ib, ob):
            @plsc.parallel_loop(0, groups, unroll=8)
            def group_body(g):
                r = g // grp_per_row
                c = (g % grp_per_row) * _LANES
                iv = ib[r, pl.ds(c, _LANES)]
                for k in range(n_cols):
                    val = plsc.load_gather(tbl, [iv + k * n_rows])
                    ob[k, r, pl.ds(c, _LANES)] = val

        idx_copy(u0, idx_a, isem_a).start()
        idx_copy(u0 + 1, idx_b, isem_b).start()
        n_pairs = units_per_w // 2

        def pair_body(j, carry):
            for p, (ib, ob, isem, osem) in enumerate(bufs):
                i = u0 + 2 * j + p
                idx_copy(i, ib, isem).wait()

                @pl.when(j > 0)
                def _():
                    out_copy(i - 2, ob, osem).wait()

                compute(ib, ob)
                out_copy(i, ob, osem).start()

                @pl.when(2 * j + p + 2 < units_per_w)
                def _():
                    idx_copy(i + 2, ib, isem).start()

            return carry

        lax.fori_loop(0, n_pairs, pair_body, 0)

        if units_per_w % 2:
            last = u0 + units_per_w - 1
            idx_copy(last, idx_a, isem_a).wait()
            out_copy(last - 2, out_a, osem_a).wait()
            compute(idx_a, out_a)
            out_copy(last, out_a, osem_a).start()
            out_copy(last - 1, out_b, osem_b).wait()
            out_copy(last, out_a, osem_a).wait()
        else:
            out_copy(u0 + units_per_w - 2, out_a, osem_a).wait()
            out_copy(u0 + units_per_w - 1, out_b, osem_b).wait()

    return lookup


def kernel(atomic_numbers, weight):
    n_batch, n_seq = atomic_numbers.shape
    n_rows, n_cols = weight.shape
    lookup = _build_sc_lookup(n_batch, n_seq, n_rows, n_cols)
    idx_t = atomic_numbers.T
    tbl_t = weight.T.reshape(n_rows * n_cols)
    out_t = lookup(idx_t, tbl_t)
    return out_t.transpose(2, 1, 0)
